# baseline (speedup 1.0000x reference)
import functools

import numpy as np
import jax
import jax.numpy as jnp
from jax import lax
from jax.experimental import pallas as pl
from jax.experimental.pallas import tpu as pltpu
from jax.experimental.pallas import tpu_sc as plsc

_VOCAB = 1000000
_EMBED = 64
_SEQ = 200
_NSEQ = 4096
_B = _NSEQ * _SEQ
_NC, _NS = 2, 16
_NW = _NC * _NS
_SEQ_PER_W = _NSEQ // _NW
_ROWS_PER_W = _B // _NW
_SCALE = 8.0
_HALF = 100


def _pe_table(length, depth):
    half = depth / 2
    positions = np.arange(length)[:, np.newaxis]
    depths = np.arange(half)[np.newaxis, :] / half
    angle_rates = 1.0 / (10000.0 ** depths)
    angle_rads = positions * angle_rates
    return np.concatenate(
        [np.sin(angle_rads), np.cos(angle_rads)], axis=-1
    ).astype(np.float32)


_PE_NP = _pe_table(_SEQ, _EMBED)


_MESH = plsc.VectorSubcoreMesh(core_axis_name="c", subcore_axis_name="s")


@functools.partial(
    pl.kernel,
    mesh=_MESH,
    out_type=jax.ShapeDtypeStruct((_B, _EMBED), jnp.float32),
    compiler_params=pltpu.CompilerParams(use_tc_tiling_on_sc=False),
    scratch_types=[
        pltpu.VMEM((_SEQ, _EMBED), jnp.float32),
        pltpu.VMEM((2, _HALF), jnp.int32),
        pltpu.VMEM((_SEQ, _EMBED), jnp.float32),
        pltpu.SemaphoreType.DMA,
    ],
)
def _emb_kernel(x_hbm, table_hbm, pe_hbm, out_hbm, pe_v, idx_v, rows_v, sem):
    wid = lax.axis_index("s") * _NC + lax.axis_index("c")
    base = wid * _ROWS_PER_W
    seq0 = wid * _SEQ_PER_W
    pltpu.sync_copy(pe_hbm, pe_v)

    def seq_body(s, carry):
        b = base + s * _SEQ
        pltpu.sync_copy(x_hbm.at[pl.ds((seq0 + s) * 2, 2)], idx_v)
        cp0 = pltpu.async_copy(
            table_hbm.at[idx_v.at[0]], rows_v.at[pl.ds(0, _HALF)], sem)
        cp1 = pltpu.async_copy(
            table_hbm.at[idx_v.at[1]], rows_v.at[pl.ds(_HALF, _HALF)], sem)
        cp0.wait()
        cp1.wait()

        def row_body(r, c):
            for q in range(_EMBED // 16):
                sl = pl.ds(q * 16, 16)
                rows_v[r, sl] = rows_v[r, sl] * _SCALE + pe_v[r, sl]
            return c

        lax.fori_loop(0, _SEQ, row_body, 0)
        pltpu.sync_copy(rows_v, out_hbm.at[pl.ds(b, _SEQ)])
        return carry

    lax.fori_loop(0, _SEQ_PER_W, seq_body, 0)


def kernel(x, table):
    x2 = x.reshape(_NSEQ * 2, _HALF).astype(jnp.int32)
    out = _emb_kernel(x2, table, jnp.asarray(_PE_NP))
    return out.reshape(_NSEQ, _SEQ, _EMBED)
